# Initial kernel scaffold; baseline (speedup 1.0000x reference)
import jax
import jax.numpy as jnp
from jax.experimental import pallas as pl


def kernel(predicted_locs, predicted_scores, boxes, labels, priors_cxcy):
    raise NotImplementedError("write your pallas kernel here")


# speedup vs baseline: 63.5529x; 63.5529x over previous
import functools

import jax
import jax.numpy as jnp
from jax import lax
from jax.experimental import pallas as pl
from jax.experimental.pallas import tpu as pltpu

_B = 16
_N = 20000
_NOBJ = 20
_NCLS = 3
_SUB = 8
_LANE = 2500


def _match_kernel(obj_ref, priors_ref, locs_ref, scores_ref, neg_ref, misc_ref):
    px0 = priors_ref[0]
    py0 = priors_ref[1]
    px1 = priors_ref[2]
    py1 = priors_ref[3]
    parea = priors_ref[4]

    p_idx = (lax.broadcasted_iota(jnp.int32, (_SUB, _LANE), 0) * _LANE
             + lax.broadcasted_iota(jnp.int32, (_SUB, _LANE), 1))

    best = None
    besti = None
    pj_list = []
    for j in range(_NOBJ):
        bx0 = obj_ref[0, 0, j]
        by0 = obj_ref[0, 1, j]
        bx1 = obj_ref[0, 2, j]
        by1 = obj_ref[0, 3, j]
        barea = obj_ref[0, 4, j]
        ltx = jnp.maximum(px0, bx0)
        lty = jnp.maximum(py0, by0)
        rbx = jnp.minimum(px1, bx1)
        rby = jnp.minimum(py1, by1)
        w = jnp.maximum(rbx - ltx, 0.0)
        h = jnp.maximum(rby - lty, 0.0)
        inter = w * h
        iou = inter / (barea + parea - inter)
        if j == 0:
            best = iou
            besti = jnp.zeros((_SUB, _LANE), jnp.int32)
        else:
            gt = iou > best
            besti = jnp.where(gt, j, besti)
            best = jnp.where(gt, iou, best)
        mx = jnp.max(iou)
        pj = jnp.min(jnp.where(iou == mx, p_idx, _N))
        pj_list.append(pj)

    for j in range(_NOBJ):
        m = p_idx == pj_list[j]
        besti = jnp.where(m, j, besti)
        best = jnp.where(m, 1.0, best)

    lab = jnp.zeros((_SUB, _LANE), jnp.float32)
    bcx = jnp.zeros((_SUB, _LANE), jnp.float32)
    bcy = jnp.zeros((_SUB, _LANE), jnp.float32)
    bw = jnp.zeros((_SUB, _LANE), jnp.float32)
    bh = jnp.zeros((_SUB, _LANE), jnp.float32)
    for j in range(_NOBJ):
        m = besti == j
        lab = jnp.where(m, obj_ref[0, 9, j], lab)
        bcx = jnp.where(m, obj_ref[0, 5, j], bcx)
        bcy = jnp.where(m, obj_ref[0, 6, j], bcy)
        bw = jnp.where(m, obj_ref[0, 7, j], bw)
        bh = jnp.where(m, obj_ref[0, 8, j], bh)

    lab = jnp.where(best < 0.5, 0.0, lab)
    posf = (lab > 0.0).astype(jnp.float32)
    n_pos = jnp.sum(posf)

    g0 = (bcx - priors_ref[5]) / priors_ref[7]
    g1 = (bcy - priors_ref[6]) / priors_ref[8]
    g2 = jnp.log(bw / priors_ref[9]) * 5.0
    g3 = jnp.log(bh / priors_ref[10]) * 5.0
    loc_sum = jnp.sum((jnp.abs(locs_ref[0, 0] - g0) + jnp.abs(locs_ref[0, 1] - g1)
                       + jnp.abs(locs_ref[0, 2] - g2) + jnp.abs(locs_ref[0, 3] - g3))
                      * posf)

    t0 = (lab == 1.0).astype(jnp.float32)
    t_bg = (lab == 0.0).astype(jnp.float32)
    t_pair = (lab == 3.0).astype(jnp.float32)
    tgt0 = t_bg
    tgt1 = t0 + t_pair
    tgt2 = (lab == 2.0).astype(jnp.float32) + t_pair
    bce = jnp.zeros((_SUB, _LANE), jnp.float32)
    for c, tgt in enumerate((tgt0, tgt1, tgt2)):
        l = scores_ref[0, c]
        bce = bce + (jnp.maximum(l, 0.0) - l * tgt + jnp.log1p(jnp.exp(-jnp.abs(l))))

    conf_pos = jnp.sum(bce * posf)
    neg_ref[0] = jnp.where(posf > 0.0, 0.0, bce)

    lane = lax.broadcasted_iota(jnp.int32, (1, 128), 1)
    misc_ref[0] = (jnp.where(lane == 0, loc_sum, 0.0)
                   + jnp.where(lane == 1, conf_pos, 0.0)
                   + jnp.where(lane == 2, n_pos, 0.0))


def _topk_kernel(neg_ref, misc_ref, total_ref, conf_ref, loc_ref):
    neg = neg_ref[...]
    bits = lax.bitcast_convert_type(neg, jnp.int32)
    misc = misc_ref[...]
    lane = lax.broadcasted_iota(jnp.int32, misc.shape, 2)
    loc_sum = jnp.sum(jnp.where(lane == 0, misc, 0.0))
    conf_pos = jnp.sum(jnp.where(lane == 1, misc, 0.0))
    nposv = jnp.sum(jnp.where(lane == 2, misc, 0.0), axis=2, keepdims=True)
    n_pos_tot = jnp.sum(nposv)
    k = jnp.minimum(nposv * 3.0, float(_N)).astype(jnp.int32)

    m = jnp.zeros((_B, 1, 1), jnp.int32)
    for bit in range(30, -1, -1):
        cand = m | (1 << bit)
        cnt = jnp.sum((bits >= cand).astype(jnp.int32), axis=(1, 2), keepdims=True)
        m = jnp.where(cnt >= k, cand, m)
    tval = lax.bitcast_convert_type(m, jnp.float32)
    gt = bits > m
    cnt_gt = jnp.sum(gt.astype(jnp.float32), axis=(1, 2), keepdims=True)
    sum_gt = jnp.sum(jnp.where(gt, neg, 0.0), axis=(1, 2), keepdims=True)
    s = sum_gt + (k.astype(jnp.float32) - cnt_gt) * tval
    s = jnp.where(k == 0, 0.0, s)
    hard = jnp.sum(s)

    conf_loss = (hard + conf_pos) / (1e-10 + n_pos_tot)
    loc_loss = loc_sum / jnp.maximum(4.0 * n_pos_tot, 1.0)
    total_ref[0, 0] = conf_loss + loc_loss
    conf_ref[0, 0] = conf_loss
    loc_ref[0, 0] = loc_loss


@functools.partial(jax.jit, static_argnames=())
def kernel(predicted_locs, predicted_scores, boxes, labels, priors_cxcy):
    pcx = priors_cxcy[:, 0]
    pcy = priors_cxcy[:, 1]
    pw = priors_cxcy[:, 2]
    ph = priors_cxcy[:, 3]
    px0 = pcx - pw / 2.0
    py0 = pcy - ph / 2.0
    px1 = pcx + pw / 2.0
    py1 = pcy + ph / 2.0
    parea = (px1 - px0) * (py1 - py0)
    priors_pl = jnp.stack(
        [px0, py0, px1, py1, parea, pcx, pcy, pw / 10.0, ph / 10.0, pw, ph]
    ).reshape(11, _SUB, _LANE)

    bx0 = boxes[:, :, 0]
    by0 = boxes[:, :, 1]
    bx1 = boxes[:, :, 2]
    by1 = boxes[:, :, 3]
    obj = jnp.stack(
        [bx0, by0, bx1, by1,
         (bx1 - bx0) * (by1 - by0),
         (bx0 + bx1) / 2.0, (by0 + by1) / 2.0,
         bx1 - bx0, by1 - by0,
         labels.astype(jnp.float32)],
        axis=1)

    locs_t = predicted_locs.transpose(0, 2, 1).reshape(_B, 4, _SUB, _LANE)
    scores_t = predicted_scores.transpose(0, 2, 1).reshape(_B, _NCLS, _SUB, _LANE)

    neg, misc = pl.pallas_call(
        _match_kernel,
        grid=(_B,),
        in_specs=[
            pl.BlockSpec((1, 10, _NOBJ), lambda b: (b, 0, 0),
                         memory_space=pltpu.SMEM),
            pl.BlockSpec((11, _SUB, _LANE), lambda b: (0, 0, 0)),
            pl.BlockSpec((1, 4, _SUB, _LANE), lambda b: (b, 0, 0, 0)),
            pl.BlockSpec((1, _NCLS, _SUB, _LANE), lambda b: (b, 0, 0, 0)),
        ],
        out_specs=[
            pl.BlockSpec((1, _SUB, _LANE), lambda b: (b, 0, 0)),
            pl.BlockSpec((1, 1, 128), lambda b: (b, 0, 0)),
        ],
        out_shape=[
            jax.ShapeDtypeStruct((_B, _SUB, _LANE), jnp.float32),
            jax.ShapeDtypeStruct((_B, 1, 128), jnp.float32),
        ],
    )(obj, priors_pl, locs_t, scores_t)

    total, conf, loc = pl.pallas_call(
        _topk_kernel,
        in_specs=[
            pl.BlockSpec((_B, _SUB, _LANE), lambda: (0, 0, 0)),
            pl.BlockSpec((_B, 1, 128), lambda: (0, 0, 0)),
        ],
        out_specs=[
            pl.BlockSpec((1, 1), lambda: (0, 0), memory_space=pltpu.SMEM),
            pl.BlockSpec((1, 1), lambda: (0, 0), memory_space=pltpu.SMEM),
            pl.BlockSpec((1, 1), lambda: (0, 0), memory_space=pltpu.SMEM),
        ],
        out_shape=[
            jax.ShapeDtypeStruct((1, 1), jnp.float32),
            jax.ShapeDtypeStruct((1, 1), jnp.float32),
            jax.ShapeDtypeStruct((1, 1), jnp.float32),
        ],
    )(neg, misc)

    n_positives = misc[:, 0, 2].astype(jnp.int32)
    return total[0, 0], conf[0, 0], loc[0, 0], n_positives
